# baseline (speedup 1.0000x reference)
import functools

import jax
import jax.numpy as jnp
from jax import lax
from jax.experimental import pallas as pl
from jax.experimental.pallas import tpu as pltpu
from jax.experimental.pallas import tpu_sc as plsc

_C = 16
_D = 4
_MAP_HW = (512, 512, 1024, 256)
_CHUNK = 256
_NW = 32
_GRP = _CHUNK // 16
_NIDX = _CHUNK * 8

_ROWS = [_D * hw * hw for hw in _MAP_HW]
_BASE = [sum(_ROWS[:i]) for i in range(4)]
_TOTAL = sum(_ROWS)


def _body(tab, uv, zc, idc, out, uvb, zb, idcb, idxb0, idxb1, wxb0, wxb1,
          wyb0, wyb1, wzb0, wzb1, texb0, texb1, outb0, outb1,
          sem0, sem1, osem0, osem1):

    idxb = (idxb0, idxb1)
    wxb = (wxb0, wxb1)
    wyb = (wyb0, wyb1)
    wzb = (wzb0, wzb1)
    texb = (texb0, texb1)
    outb = (outb0, outb1)
    sems = (sem0, sem1)
    osems = (osem0, osem1)
    nrays = out.shape[1]
    per_w = nrays // _NW
    nchunk = per_w // _CHUNK
    wid = lax.axis_index("s") * 2 + lax.axis_index("c")
    w0 = pl.multiple_of(wid * per_w, _CHUNK)


    pltpu.sync_copy(uv.at[pl.ds(2 * w0, 2 * per_w)], uvb)
    pltpu.sync_copy(zc.at[pl.ds(w0, per_w)], zb)
    pltpu.sync_copy(idc.at[pl.ds(w0, per_w)], idcb)

    def pass1(ch, idx_r, wx_r, wy_r, wz_r):
        cbase = pl.multiple_of(ch * _CHUNK, _CHUNK)

        def grp(g, c1):
            off = pl.multiple_of(g * 16, 16)
            rows = cbase + off + lax.iota(jnp.int32, 16)
            rows2 = rows + rows
            x = plsc.load_gather(uvb, [rows2])
            y = plsc.load_gather(uvb, [rows2 + 1])
            z = zb[pl.ds(cbase + off, 16)]
            mi = idcb[pl.ds(cbase + off, 16)]
            is2 = mi == 2
            is3 = mi == 3
            wf = jnp.where(is2, 1024.0, jnp.where(is3, 256.0, 512.0))
            wi = jnp.where(is2, 1024, jnp.where(is3, 256, 512))
            base = jnp.where(mi == 0, _BASE[0],
                             jnp.where(mi == 1, _BASE[1],
                                       jnp.where(is2, _BASE[2], _BASE[3])))
            wm1 = wf - 1.0
            ix = jnp.clip((x + 1.0) * 0.5 * wm1, 0.0, wm1)
            iy = jnp.clip((y + 1.0) * 0.5 * wm1, 0.0, wm1)
            iz = jnp.clip((z + 1.0) * (0.5 * (_D - 1)), 0.0, float(_D - 1))
            x0 = ix.astype(jnp.int32)
            y0 = iy.astype(jnp.int32)
            z0 = iz.astype(jnp.int32)
            wx = ix - x0.astype(jnp.float32)
            wy = iy - y0.astype(jnp.float32)
            wz = iz - z0.astype(jnp.float32)
            x1 = jnp.minimum(x0 + 1, wi - 1)
            y1 = jnp.minimum(y0 + 1, wi - 1)
            z1 = jnp.minimum(z0 + 1, _D - 1)
            hw = wi * wi
            b00 = base + z0 * hw + y0 * wi
            b01 = base + z0 * hw + y1 * wi
            b10 = base + z1 * hw + y0 * wi
            b11 = base + z1 * hw + y1 * wi
            corners = (b00 + x0, b00 + x1, b01 + x0, b01 + x1,
                       b10 + x0, b10 + x1, b11 + x0, b11 + x1)
            for k in range(8):
                idx_r[pl.ds(off + k * _CHUNK, 16)] = corners[k]
            wx_r[pl.ds(off, 16)] = wx
            wy_r[pl.ds(off, 16)] = wy
            wz_r[pl.ds(off, 16)] = wz
            return c1

        lax.fori_loop(0, _GRP, grp, 0, unroll=False)

    def fire(idx_r, tex_r, sem):
        for j in range(_NIDX // 128):
            sl = pl.ds(j * 128, 128)
            pltpu.async_copy(tab.at[idx_r.at[sl]], tex_r.at[sl, :], sem)

    def drain(tex_r, sem):
        pltpu.make_async_copy(tab.at[pl.ds(0, _NIDX), :], tex_r, sem).wait()

    def pass2(tex_r, wx_r, wy_r, wz_r, out_r):
        def grp(g, c1):
            off = pl.multiple_of(g * 16, 16)
            rows = off + lax.iota(jnp.int32, 16)
            wx = wx_r[pl.ds(off, 16)]
            wy = wy_r[pl.ds(off, 16)]
            wz = wz_r[pl.ds(off, 16)]
            u0 = 1.0 - wx
            v0 = 1.0 - wy
            t0 = 1.0 - wz
            tv00 = t0 * v0
            tv01 = t0 * wy
            tv10 = wz * v0
            tv11 = wz * wy
            w = (tv00 * u0, tv00 * wx, tv01 * u0, tv01 * wx,
                 tv10 * u0, tv10 * wx, tv11 * u0, tv11 * wx)
            rk = [rows + k * _CHUNK for k in range(8)]
            for p in range(_C // 2):
                cc = jnp.full((16,), p, jnp.int32)
                acc0 = None
                acc1 = None
                for k in range(8):
                    v = plsc.load_gather(tex_r, [rk[k], cc])
                    a, b = plsc.unpack(plsc.bitcast(v, jnp.bfloat16),
                                       format=plsc.PackFormat.INTERLEAVED,
                                       preferred_element_type=jnp.float32)
                    if k == 0:
                        acc0 = w[0] * a
                        acc1 = w[0] * b
                    else:
                        acc0 = acc0 + w[k] * a
                        acc1 = acc1 + w[k] * b
                out_r[2 * p, pl.ds(off, 16)] = acc0
                out_r[2 * p + 1, pl.ds(off, 16)] = acc1
            return c1

        lax.fori_loop(0, _GRP, grp, 0, unroll=False)

    def out_dma(ch, out_r, osem):
        cbase = pl.multiple_of(w0 + ch * _CHUNK, _CHUNK)
        return pltpu.make_async_copy(out_r, out.at[:, pl.ds(cbase, _CHUNK)],
                                     osem)

    pass1(0, idxb[0], wxb[0], wyb[0], wzb[0])
    fire(idxb[0], texb[0], sems[0])

    def half(i, ch, p, q):
        @pl.when(ch + 1 < nchunk)
        def _():
            pass1(ch + 1, idxb[q], wxb[q], wyb[q], wzb[q])
            fire(idxb[q], texb[q], sems[q])

        drain(texb[p], sems[p])

        @pl.when(i > 0)
        def _():
            out_dma(ch - 2, outb[p], osems[p]).wait()

        pass2(texb[p], wxb[p], wyb[p], wzb[p], outb[p])
        out_dma(ch, outb[p], osems[p]).start()

    def pair(i, carry):
        half(i, 2 * i, 0, 1)
        half(i, 2 * i + 1, 1, 0)
        return carry

    lax.fori_loop(0, nchunk // 2, pair, 0, unroll=False)
    out_dma(nchunk - 2, outb[0], osems[0]).wait()
    out_dma(nchunk - 1, outb[1], osems[1]).wait()


def _sample(tab, uv, zc, idc):
    r = zc.shape[0]
    per_w = r // _NW
    f = pl.kernel(
        _body,
        out_type=jax.ShapeDtypeStruct((_C, r), jnp.float32),
        mesh=plsc.VectorSubcoreMesh(core_axis_name="c", subcore_axis_name="s"),
        compiler_params=pltpu.CompilerParams(needs_layout_passes=False,
                                             use_tc_tiling_on_sc=False),
        scratch_types=[
            pltpu.VMEM((2 * per_w,), jnp.float32),
            pltpu.VMEM((per_w,), jnp.float32),
            pltpu.VMEM((per_w,), jnp.int32),
            pltpu.VMEM((_NIDX,), jnp.int32),
            pltpu.VMEM((_NIDX,), jnp.int32),
            pltpu.VMEM((_CHUNK,), jnp.float32),
            pltpu.VMEM((_CHUNK,), jnp.float32),
            pltpu.VMEM((_CHUNK,), jnp.float32),
            pltpu.VMEM((_CHUNK,), jnp.float32),
            pltpu.VMEM((_CHUNK,), jnp.float32),
            pltpu.VMEM((_CHUNK,), jnp.float32),
            pltpu.VMEM((_NIDX, _C // 2), jnp.int32),
            pltpu.VMEM((_NIDX, _C // 2), jnp.int32),
            pltpu.VMEM((_C, _CHUNK), jnp.float32),
            pltpu.VMEM((_C, _CHUNK), jnp.float32),
            pltpu.SemaphoreType.DMA,
            pltpu.SemaphoreType.DMA,
            pltpu.SemaphoreType.DMA,
            pltpu.SemaphoreType.DMA,
        ],
    )
    return f(tab, uv, zc, idc)


def kernel(uv_coords, uv_idcs, subject_id, map0, map1, map2, map3):
    maps = (map0, map1, map2, map3)
    tab = jnp.concatenate(
        [jnp.transpose(m.astype(jnp.bfloat16), (0, 2, 3, 1)).reshape(-1, _C)
         for m in maps], axis=0)
    tab_i = jax.lax.bitcast_convert_type(
        tab.reshape(_TOTAL, _C // 2, 2), jnp.int32)
    idc = uv_idcs.astype(jnp.int32)
    return _sample(tab_i, uv_coords.reshape(-1), subject_id, idc)


# speedup vs baseline: 1.3816x; 1.3816x over previous
import functools

import jax
import jax.numpy as jnp
from jax import lax
from jax.experimental import pallas as pl
from jax.experimental.pallas import tpu as pltpu
from jax.experimental.pallas import tpu_sc as plsc

_C = 16
_D = 4
_MAP_HW = (512, 512, 1024, 256)
_CHUNK = 256
_NW = 32
_GRP = _CHUNK // 16
_NIDX = _CHUNK * 8

_ROWS = [_D * hw * hw for hw in _MAP_HW]
_BASE = [sum(_ROWS[:i]) for i in range(4)]
_TOTAL = sum(_ROWS)


def _body(tab, uv, zc, idc, out, uvb, zb, idcb, idxb0, idxb1, wxb0, wxb1,
          wyb0, wyb1, wzb0, wzb1, texb0, texb1, outb0, outb1,
          sem0, sem1, osem0, osem1):

    idxb = (idxb0, idxb1)
    wxb = (wxb0, wxb1)
    wyb = (wyb0, wyb1)
    wzb = (wzb0, wzb1)
    texb = (texb0, texb1)
    outb = (outb0, outb1)
    sems = (sem0, sem1)
    osems = (osem0, osem1)
    nrays = out.shape[1]
    per_w = nrays // _NW
    nchunk = per_w // _CHUNK
    wid = lax.axis_index("s") * 2 + lax.axis_index("c")
    w0 = pl.multiple_of(wid * per_w, _CHUNK)


    pltpu.sync_copy(uv.at[pl.ds(2 * w0, 2 * per_w)], uvb)
    pltpu.sync_copy(zc.at[pl.ds(w0, per_w)], zb)
    pltpu.sync_copy(idc.at[pl.ds(w0, per_w)], idcb)

    def pass1(ch, idx_r, wx_r, wy_r, wz_r):
        cbase = pl.multiple_of(ch * _CHUNK, _CHUNK)

        def grp(g, c1):
            off = pl.multiple_of(g * 16, 16)
            rows = cbase + off + lax.iota(jnp.int32, 16)
            rows2 = rows + rows
            x = plsc.load_gather(uvb, [rows2])
            y = plsc.load_gather(uvb, [rows2 + 1])
            z = zb[pl.ds(cbase + off, 16)]
            mi = idcb[pl.ds(cbase + off, 16)]
            is2 = mi == 2
            is3 = mi == 3
            wf = jnp.where(is2, 1024.0, jnp.where(is3, 256.0, 512.0))
            wi = jnp.where(is2, 1024, jnp.where(is3, 256, 512))
            base = jnp.where(mi == 0, _BASE[0],
                             jnp.where(mi == 1, _BASE[1],
                                       jnp.where(is2, _BASE[2], _BASE[3])))
            wm1 = wf - 1.0
            ix = jnp.clip((x + 1.0) * 0.5 * wm1, 0.0, wm1)
            iy = jnp.clip((y + 1.0) * 0.5 * wm1, 0.0, wm1)
            iz = jnp.clip((z + 1.0) * (0.5 * (_D - 1)), 0.0, float(_D - 1))
            x0 = ix.astype(jnp.int32)
            y0 = iy.astype(jnp.int32)
            z0 = iz.astype(jnp.int32)
            wx = ix - x0.astype(jnp.float32)
            wy = iy - y0.astype(jnp.float32)
            wz = iz - z0.astype(jnp.float32)
            x1 = jnp.minimum(x0 + 1, wi - 1)
            y1 = jnp.minimum(y0 + 1, wi - 1)
            z1 = jnp.minimum(z0 + 1, _D - 1)
            hw = wi * wi
            b00 = base + z0 * hw + y0 * wi
            b01 = base + z0 * hw + y1 * wi
            b10 = base + z1 * hw + y0 * wi
            b11 = base + z1 * hw + y1 * wi
            corners = (b00 + x0, b00 + x1, b01 + x0, b01 + x1,
                       b10 + x0, b10 + x1, b11 + x0, b11 + x1)
            for k in range(8):
                idx_r[pl.ds(off + k * _CHUNK, 16)] = corners[k]
            wx_r[pl.ds(off, 16)] = wx
            wy_r[pl.ds(off, 16)] = wy
            wz_r[pl.ds(off, 16)] = wz
            return c1

        lax.fori_loop(0, _GRP, grp, 0, unroll=False)

    def fire(idx_r, tex_r, sem):
        for j in range(_NIDX // 128):
            sl = pl.ds(j * 128, 128)
            pltpu.async_copy(tab.at[idx_r.at[sl]], tex_r.at[sl, :], sem)

    def drain(tex_r, sem):
        pltpu.make_async_copy(tab.at[pl.ds(0, _NIDX), :], tex_r, sem).wait()

    def pass2(tex_r, wx_r, wy_r, wz_r, out_r):
        def grp(g, c1):
            off = pl.multiple_of(g * 16, 16)
            rows = off + lax.iota(jnp.int32, 16)
            wx = wx_r[pl.ds(off, 16)]
            wy = wy_r[pl.ds(off, 16)]
            wz = wz_r[pl.ds(off, 16)]
            u0 = 1.0 - wx
            v0 = 1.0 - wy
            t0 = 1.0 - wz
            tv00 = t0 * v0
            tv01 = t0 * wy
            tv10 = wz * v0
            tv11 = wz * wy
            w = (tv00 * u0, tv00 * wx, tv01 * u0, tv01 * wx,
                 tv10 * u0, tv10 * wx, tv11 * u0, tv11 * wx)
            rk = [rows + k * _CHUNK for k in range(8)]
            for c in range(_C):
                cc = jnp.full((16,), c, jnp.int32)
                acc = w[0] * plsc.load_gather(tex_r, [rk[0], cc])
                for k in range(1, 8):
                    acc = acc + w[k] * plsc.load_gather(tex_r, [rk[k], cc])
                out_r[c, pl.ds(off, 16)] = acc
            return c1

        lax.fori_loop(0, _GRP, grp, 0, unroll=False)

    def out_dma(ch, out_r, osem):
        cbase = pl.multiple_of(w0 + ch * _CHUNK, _CHUNK)
        return pltpu.make_async_copy(out_r, out.at[:, pl.ds(cbase, _CHUNK)],
                                     osem)

    pass1(0, idxb[0], wxb[0], wyb[0], wzb[0])
    fire(idxb[0], texb[0], sems[0])

    def half(i, ch, p, q):
        @pl.when(ch + 1 < nchunk)
        def _():
            pass1(ch + 1, idxb[q], wxb[q], wyb[q], wzb[q])
            fire(idxb[q], texb[q], sems[q])

        drain(texb[p], sems[p])

        @pl.when(i > 0)
        def _():
            out_dma(ch - 2, outb[p], osems[p]).wait()

        pass2(texb[p], wxb[p], wyb[p], wzb[p], outb[p])
        out_dma(ch, outb[p], osems[p]).start()

    def pair(i, carry):
        half(i, 2 * i, 0, 1)
        half(i, 2 * i + 1, 1, 0)
        return carry

    lax.fori_loop(0, nchunk // 2, pair, 0, unroll=False)
    out_dma(nchunk - 2, outb[0], osems[0]).wait()
    out_dma(nchunk - 1, outb[1], osems[1]).wait()


def _sample(tab, uv, zc, idc):
    r = zc.shape[0]
    per_w = r // _NW
    f = pl.kernel(
        _body,
        out_type=jax.ShapeDtypeStruct((_C, r), jnp.float32),
        mesh=plsc.VectorSubcoreMesh(core_axis_name="c", subcore_axis_name="s"),
        compiler_params=pltpu.CompilerParams(needs_layout_passes=False,
                                             use_tc_tiling_on_sc=False),
        scratch_types=[
            pltpu.VMEM((2 * per_w,), jnp.float32),
            pltpu.VMEM((per_w,), jnp.float32),
            pltpu.VMEM((per_w,), jnp.int32),
            pltpu.VMEM((_NIDX,), jnp.int32),
            pltpu.VMEM((_NIDX,), jnp.int32),
            pltpu.VMEM((_CHUNK,), jnp.float32),
            pltpu.VMEM((_CHUNK,), jnp.float32),
            pltpu.VMEM((_CHUNK,), jnp.float32),
            pltpu.VMEM((_CHUNK,), jnp.float32),
            pltpu.VMEM((_CHUNK,), jnp.float32),
            pltpu.VMEM((_CHUNK,), jnp.float32),
            pltpu.VMEM((_NIDX, _C), jnp.float32),
            pltpu.VMEM((_NIDX, _C), jnp.float32),
            pltpu.VMEM((_C, _CHUNK), jnp.float32),
            pltpu.VMEM((_C, _CHUNK), jnp.float32),
            pltpu.SemaphoreType.DMA,
            pltpu.SemaphoreType.DMA,
            pltpu.SemaphoreType.DMA,
            pltpu.SemaphoreType.DMA,
        ],
    )
    return f(tab, uv, zc, idc)


def kernel(uv_coords, uv_idcs, subject_id, map0, map1, map2, map3):
    maps = (map0, map1, map2, map3)
    tab = jnp.concatenate(
        [jnp.transpose(m, (0, 2, 3, 1)).reshape(-1, _C) for m in maps],
        axis=0)
    idc = uv_idcs.astype(jnp.int32)
    return _sample(tab, uv_coords.reshape(-1), subject_id, idc)
